# baseline (speedup 1.0000x reference)
import functools

import jax
import jax.numpy as jnp
from jax.experimental import pallas as pl
from jax.experimental.pallas import tpu as pltpu

G = 512
C = 128
W = 128
F32 = jnp.float32


def _pick_blk(n):
    for b in (2000, 1600, 1000, 800, 400, 200, 100, 50, 40, 25, 20, 10, 8, 5, 4, 2, 1):
        if n % b == 0 and (b % 8 == 0 or b == n):
            return b
    return n


def _body(nb, x_op, x_ma, x_ag, b_op, b_ma, b_ag,
          tok_op, wl_op, bl_op, wr_op, br_op, att_op, bias_op,
          tok_ma, wl_ma, bl_ma, wr_ma, br_ma, att_ma, bias_ma,
          tok_ag, wl_ag, bl_ag, wr_ag, br_ag, att_ag, bias_ag,
          mw1, mb1, mw2, mb2,
          o_op, o_ma, o_ag, o_gf,
          num_op, den_op, num_ma, den_ma, num_ag, den_ag):
    i = pl.program_id(0)
    types = (
        (x_op, b_op, tok_op, wl_op, bl_op, wr_op, br_op, att_op, num_op, den_op),
        (x_ma, b_ma, tok_ma, wl_ma, bl_ma, wr_ma, br_ma, att_ma, num_ma, den_ma),
        (x_ag, b_ag, tok_ag, wl_ag, bl_ag, wr_ag, br_ag, att_ag, num_ag, den_ag),
    )

    @pl.when(i == 0)
    def _init():
        for (_, _, _, _, _, _, _, _, num, den) in types:
            num[...] = jnp.zeros_like(num)
            den[...] = jnp.zeros_like(den)

    for (x, b, tok, wl, bl, wr, br, att, num, den) in types:
        xb = x[...]
        blk = xb.shape[0]
        gl = jax.lax.dot_general(xb, wl[...], (((1,), (1,)), ((), ())),
                                 preferred_element_type=F32) + bl[...]
        gr = jax.lax.dot_general(tok[...], wr[...], (((1,), (1,)), ((), ())),
                                 preferred_element_type=F32) + br[...]
        e = gl + gr
        e = jnp.where(e >= 0, e, 0.2 * e)
        score = jax.lax.dot_general(e, att[...], (((1,), (0,)), ((), ())),
                                    preferred_element_type=F32)
        w = jnp.exp(score)
        wt = w.reshape(1, blk)
        bb = b[...].reshape(1, blk)
        bfirst = jnp.min(bb)
        blast = jnp.max(bb)
        base = jnp.minimum((bfirst // 8) * 8, G - W)
        span_ok = (blast - base) < W

        @pl.when(span_ok)
        def _narrow():
            rel = bb - base
            ohw = jnp.where(jax.lax.broadcasted_iota(jnp.int32, (W, blk), 0)
                            == rel, wt, 0.0)
            dnum = jax.lax.dot_general(ohw, gl, (((1,), (0,)), ((), ())),
                                       preferred_element_type=F32)
            dden = jnp.sum(ohw, axis=1, keepdims=True)
            sl = pl.ds(pl.multiple_of(base, 8), W)
            num[sl, :] += dnum
            den[sl, :] += dden

        @pl.when(jnp.logical_not(span_ok))
        def _full():
            seg = jax.lax.broadcasted_iota(jnp.int32, (G, blk), 0)
            oh = jnp.where(seg == bb, wt, 0.0)
            num[...] += jax.lax.dot_general(oh, gl, (((1,), (0,)), ((), ())),
                                            preferred_element_type=F32)
            den[...] += jnp.sum(oh, axis=1, keepdims=True)

    @pl.when(i == nb - 1)
    def _finish():
        gs = []
        for (_, _, _, _, _, _, _, _, num, den), bias, out in (
                (types[0], bias_op, o_op), (types[1], bias_ma, o_ma),
                (types[2], bias_ag, o_ag)):
            g = num[...] / jnp.maximum(den[...], 1e-16) + bias[...]
            out[...] = g
            gs.append(g)
        h = jnp.concatenate(gs, axis=1)
        h = jax.lax.dot_general(h, mw1[...], (((1,), (1,)), ((), ())),
                                preferred_element_type=F32) + mb1[...]
        h = jnp.where(h >= 0, h, 0.01 * h)
        gf = jax.lax.dot_general(h, mw2[...], (((1,), (1,)), ((), ())),
                                 preferred_element_type=F32) + mb2[...]
        o_gf[...] = gf


def kernel(x_operation, x_machine, x_AGV, batch_operation, batch_machine, batch_AGV,
           token_operation, Wl_operation, bl_operation, Wr_operation, br_operation,
           att_operation, bias_operation,
           token_machine, Wl_machine, bl_machine, Wr_machine, br_machine,
           att_machine, bias_machine,
           token_AGV, Wl_AGV, bl_AGV, Wr_AGV, br_AGV, att_AGV, bias_AGV,
           mix_W1, mix_b1, mix_W2, mix_b2):
    n = x_operation.shape[0]
    blk = _pick_blk(n)
    nb = n // blk
    gg = mix_W1.shape[0]

    row2 = lambda v: v.reshape(1, -1)
    col2 = lambda v: v.reshape(-1, 1)
    b3 = lambda b: b.reshape(nb, 1, blk)

    x_spec = pl.BlockSpec((blk, C), lambda i: (i, 0))
    b_spec = pl.BlockSpec((1, 1, blk), lambda i: (i, 0, 0))
    full2 = lambda a: pl.BlockSpec(a.shape, lambda i: (0, 0))

    params = []
    specs = [x_spec, x_spec, x_spec, b_spec, b_spec, b_spec]
    for tok, wl, bl, wr, br, att, bias in (
            (token_operation, Wl_operation, bl_operation, Wr_operation, br_operation,
             att_operation, bias_operation),
            (token_machine, Wl_machine, bl_machine, Wr_machine, br_machine,
             att_machine, bias_machine),
            (token_AGV, Wl_AGV, bl_AGV, Wr_AGV, br_AGV, att_AGV, bias_AGV)):
        args = (row2(tok), wl, row2(bl), wr, row2(br), col2(att), row2(bias))
        params.extend(args)
        specs.extend(full2(a) for a in args)
    mix = (mix_W1, row2(mix_b1), mix_W2, row2(mix_b2))
    params.extend(mix)
    specs.extend(full2(a) for a in mix)

    out_shape = (
        jax.ShapeDtypeStruct((G, C), F32),
        jax.ShapeDtypeStruct((G, C), F32),
        jax.ShapeDtypeStruct((G, C), F32),
        jax.ShapeDtypeStruct((G, gg), F32),
    )
    out_specs = (
        pl.BlockSpec((G, C), lambda i: (0, 0)),
        pl.BlockSpec((G, C), lambda i: (0, 0)),
        pl.BlockSpec((G, C), lambda i: (0, 0)),
        pl.BlockSpec((G, gg), lambda i: (0, 0)),
    )
    scratch = []
    for _ in range(3):
        scratch.append(pltpu.VMEM((G, C), F32))
        scratch.append(pltpu.VMEM((G, 1), F32))

    return pl.pallas_call(
        functools.partial(_body, nb),
        grid=(nb,),
        in_specs=specs,
        out_specs=out_specs,
        out_shape=out_shape,
        scratch_shapes=scratch,
        compiler_params=pltpu.CompilerParams(
            dimension_semantics=("arbitrary",),
        ),
    )(x_operation, x_machine, x_AGV,
      b3(batch_operation), b3(batch_machine), b3(batch_AGV), *params)


# speedup vs baseline: 1.0443x; 1.0443x over previous
import functools

import jax
import jax.numpy as jnp
from jax.experimental import pallas as pl
from jax.experimental.pallas import tpu as pltpu

G = 512
C = 128
W = 128
F32 = jnp.float32
BF16 = jnp.bfloat16


def _pick_blk(n):
    for b in (2000, 1600, 1000, 800, 400, 200, 100, 50, 40, 25, 20, 10, 8, 5, 4, 2, 1):
        if n % b == 0 and (b % 8 == 0 or b == n):
            return b
    return n


def _body(nb, x_op, x_ma, x_ag, b_op, b_ma, b_ag,
          tok_op, wl_op, bl_op, wr_op, br_op, att_op, bias_op,
          tok_ma, wl_ma, bl_ma, wr_ma, br_ma, att_ma, bias_ma,
          tok_ag, wl_ag, bl_ag, wr_ag, br_ag, att_ag, bias_ag,
          mw1, mb1, mw2, mb2,
          o_op, o_ma, o_ag, o_gf,
          num_op, den_op, num_ma, den_ma, num_ag, den_ag):
    i = pl.program_id(0)
    types = (
        (x_op, b_op, tok_op, wl_op, bl_op, wr_op, br_op, att_op, num_op, den_op),
        (x_ma, b_ma, tok_ma, wl_ma, bl_ma, wr_ma, br_ma, att_ma, num_ma, den_ma),
        (x_ag, b_ag, tok_ag, wl_ag, bl_ag, wr_ag, br_ag, att_ag, num_ag, den_ag),
    )

    @pl.when(i == 0)
    def _init():
        for (_, _, _, _, _, _, _, _, num, den) in types:
            num[...] = jnp.zeros_like(num)
            den[...] = jnp.zeros_like(den)

    for (x, b, tok, wl, bl, wr, br, att, num, den) in types:
        xb = x[...]
        blk = xb.shape[0]
        gl = jax.lax.dot_general(xb, wl[...], (((1,), (1,)), ((), ())),
                                 preferred_element_type=F32) + bl[...]
        gr = jax.lax.dot_general(tok[...], wr[...], (((1,), (1,)), ((), ())),
                                 preferred_element_type=F32) + br[...]
        e = gl + gr
        e = jnp.where(e >= 0, e, 0.2 * e)
        score = jax.lax.dot_general(e, att[...], (((1,), (0,)), ((), ())),
                                    preferred_element_type=F32)
        w = jnp.exp(score)
        bb = b[...].reshape(1, blk)
        wgl = (gl * w).astype(BF16)
        wb = w.astype(BF16)
        bfirst = jnp.min(bb)
        blast = jnp.max(bb)
        base = jnp.minimum((bfirst // 8) * 8, G - W)
        span_ok = (blast - base) < W

        @pl.when(span_ok)
        def _narrow():
            rel = bb - base
            ohw = (jax.lax.broadcasted_iota(jnp.int32, (W, blk), 0)
                   == rel).astype(BF16)
            dnum = jax.lax.dot_general(ohw, wgl, (((1,), (0,)), ((), ())),
                                       preferred_element_type=F32)
            dden = jax.lax.dot_general(ohw, wb, (((1,), (0,)), ((), ())),
                                       preferred_element_type=F32)
            sl = pl.ds(pl.multiple_of(base, 8), W)
            num[sl, :] += dnum
            den[sl, :] += dden

        @pl.when(jnp.logical_not(span_ok))
        def _full():
            seg = jax.lax.broadcasted_iota(jnp.int32, (G, blk), 0)
            oh = (seg == bb).astype(BF16)
            num[...] += jax.lax.dot_general(oh, wgl, (((1,), (0,)), ((), ())),
                                            preferred_element_type=F32)
            den[...] += jax.lax.dot_general(oh, wb, (((1,), (0,)), ((), ())),
                                            preferred_element_type=F32)

    @pl.when(i == nb - 1)
    def _finish():
        gs = []
        for (_, _, _, _, _, _, _, _, num, den), bias, out in (
                (types[0], bias_op, o_op), (types[1], bias_ma, o_ma),
                (types[2], bias_ag, o_ag)):
            g = num[...] / jnp.maximum(den[...], 1e-16) + bias[...]
            out[...] = g
            gs.append(g)
        h = jnp.concatenate(gs, axis=1)
        h = jax.lax.dot_general(h, mw1[...], (((1,), (1,)), ((), ())),
                                preferred_element_type=F32) + mb1[...]
        h = jnp.where(h >= 0, h, 0.01 * h)
        gf = jax.lax.dot_general(h, mw2[...], (((1,), (1,)), ((), ())),
                                 preferred_element_type=F32) + mb2[...]
        o_gf[...] = gf


def kernel(x_operation, x_machine, x_AGV, batch_operation, batch_machine, batch_AGV,
           token_operation, Wl_operation, bl_operation, Wr_operation, br_operation,
           att_operation, bias_operation,
           token_machine, Wl_machine, bl_machine, Wr_machine, br_machine,
           att_machine, bias_machine,
           token_AGV, Wl_AGV, bl_AGV, Wr_AGV, br_AGV, att_AGV, bias_AGV,
           mix_W1, mix_b1, mix_W2, mix_b2):
    n = x_operation.shape[0]
    blk = _pick_blk(n)
    nb = n // blk
    gg = mix_W1.shape[0]

    row2 = lambda v: v.reshape(1, -1)
    col2 = lambda v: v.reshape(-1, 1)
    b3 = lambda b: b.reshape(nb, 1, blk)

    x_spec = pl.BlockSpec((blk, C), lambda i: (i, 0))
    b_spec = pl.BlockSpec((1, 1, blk), lambda i: (i, 0, 0))
    full2 = lambda a: pl.BlockSpec(a.shape, lambda i: (0, 0))

    params = []
    specs = [x_spec, x_spec, x_spec, b_spec, b_spec, b_spec]
    for tok, wl, bl, wr, br, att, bias in (
            (token_operation, Wl_operation, bl_operation, Wr_operation, br_operation,
             att_operation, bias_operation),
            (token_machine, Wl_machine, bl_machine, Wr_machine, br_machine,
             att_machine, bias_machine),
            (token_AGV, Wl_AGV, bl_AGV, Wr_AGV, br_AGV, att_AGV, bias_AGV)):
        args = (row2(tok), wl, row2(bl), wr, row2(br), col2(att), row2(bias))
        params.extend(args)
        specs.extend(full2(a) for a in args)
    mix = (mix_W1, row2(mix_b1), mix_W2, row2(mix_b2))
    params.extend(mix)
    specs.extend(full2(a) for a in mix)

    out_shape = (
        jax.ShapeDtypeStruct((G, C), F32),
        jax.ShapeDtypeStruct((G, C), F32),
        jax.ShapeDtypeStruct((G, C), F32),
        jax.ShapeDtypeStruct((G, gg), F32),
    )
    out_specs = (
        pl.BlockSpec((G, C), lambda i: (0, 0)),
        pl.BlockSpec((G, C), lambda i: (0, 0)),
        pl.BlockSpec((G, C), lambda i: (0, 0)),
        pl.BlockSpec((G, gg), lambda i: (0, 0)),
    )
    scratch = []
    for _ in range(3):
        scratch.append(pltpu.VMEM((G, C), F32))
        scratch.append(pltpu.VMEM((G, 1), F32))

    return pl.pallas_call(
        functools.partial(_body, nb),
        grid=(nb,),
        in_specs=specs,
        out_specs=out_specs,
        out_shape=out_shape,
        scratch_shapes=scratch,
        compiler_params=pltpu.CompilerParams(
            dimension_semantics=("arbitrary",),
        ),
    )(x_operation, x_machine, x_AGV,
      b3(batch_operation), b3(batch_machine), b3(batch_AGV), *params)
